# baseline (speedup 1.0000x reference)
import functools

import jax
import jax.numpy as jnp
from jax import lax
from jax.experimental import pallas as pl
from jax.experimental.pallas import tpu as pltpu
from jax.experimental.pallas import tpu_sc as plsc

NC = 2
NS = 16
L = 16
NW = NC * NS

CE = 4


@functools.lru_cache(maxsize=None)
def _build(B, S, D, V):
    assert B % NW == 0 and D % L == 0
    bpw = B // NW
    nchunks = bpw // CE
    assert bpw % CE == 0
    inv_s = 1.0 / S

    mesh = plsc.VectorSubcoreMesh(
        core_axis_name="c", subcore_axis_name="s", num_cores=NC,
        num_subcores=NS)

    @functools.partial(
        pl.kernel,
        out_type=jax.ShapeDtypeStruct((B, D), jnp.float32),
        mesh=mesh,
        scratch_types=[
            pltpu.VMEM((bpw * S,), jnp.int32),
            pltpu.VMEM((CE * S, D), jnp.float32),
            pltpu.VMEM((CE, D), jnp.float32),
            pltpu.SemaphoreType.DMA,
        ],
    )
    def emb_kernel(texts_h, emb_h, out_h, idx_v, rows_v, outb_v, gsem):
        w = lax.axis_index("s") * NC + lax.axis_index("c")
        base = w * bpw
        pltpu.sync_copy(texts_h.at[pl.ds(base * S, bpw * S)], idx_v)

        def chunk_body(g, carry):
            pltpu.async_copy(
                emb_h.at[idx_v.at[pl.ds(g * (CE * S), CE * S)]],
                rows_v, gsem).wait()
            for e in range(CE):
                for c in range(D // L):
                    ds = pl.ds(c * L, L)
                    acc = rows_v[e * S, ds]
                    for r in range(1, S):
                        acc = acc + rows_v[e * S + r, ds]
                    outb_v[e, ds] = acc * inv_s
            pltpu.sync_copy(outb_v, out_h.at[pl.ds(base + g * CE, CE)])
            return carry

        lax.fori_loop(0, nchunks, chunk_body, 0)

    return emb_kernel


def kernel(texts, emb):
    B, S = texts.shape
    V, D = emb.shape
    texts_flat = texts.reshape(-1).astype(jnp.int32)
    return _build(B, S, D, V)(texts_flat, emb)
